# Initial kernel scaffold; baseline (speedup 1.0000x reference)
import jax
import jax.numpy as jnp
from jax.experimental import pallas as pl


def kernel(x, edge_index, edge_attr, skip, W1, root1, b1, W2, root2, b2):
    raise NotImplementedError("write your pallas kernel here")


# speedup vs baseline: 2.6291x; 2.6291x over previous
import functools

import jax
import jax.numpy as jnp
from jax import lax
from jax.experimental import pallas as pl
from jax.experimental.pallas import tpu as pltpu
from jax.experimental.pallas import tpu_sc as plsc

N = 10000
E = 320000
IN = 128
OUT = 64
KS = 3
K = KS * KS


def _basis_planes(a0, a1):
    def comp(u):
        planes = []
        for j in range(KS):
            t = u - jnp.float32(j)
            at = jnp.abs(t)
            b = jnp.where(at <= 0.5, 0.75 - t * t,
                          jnp.where(at <= 1.5, 0.5 * (1.5 - at) ** 2,
                                    jnp.float32(0.0)))
            planes.append(b)
        s = planes[0] + planes[1] + planes[2] + jnp.float32(1e-12)
        return [p / s for p in planes]

    b0 = comp(a0 * jnp.float32(KS - 1))
    b1 = comp(a1 * jnp.float32(KS - 1))
    return [b0[i] * b1[j] for i in range(KS) for j in range(KS)]


def _basis_kernel(a0_ref, a1_ref, *out_refs):
    planes = _basis_planes(a0_ref[...], a1_ref[...])
    for r, p in zip(out_refs, planes):
        r[...] = p


def _compute_basis(edge_attr):
    rows = E // 128
    a0 = edge_attr[:, 0].reshape(rows, 128)
    a1 = edge_attr[:, 1].reshape(rows, 128)
    outs = pl.pallas_call(
        _basis_kernel,
        out_shape=[jax.ShapeDtypeStruct((rows, 128), jnp.float32)] * K,
    )(a0, a1)
    z = jnp.zeros_like(outs[0])
    return jnp.stack(list(outs) + [z] * (16 - K), axis=-1).reshape(E, 16)


def _mm_kernel(x_ref, w_ref, b_ref, o_ref):
    o_ref[...] = jax.lax.dot_general(
        x_ref[...], w_ref[...], (((1,), (0,)), ((), ())),
        preferred_element_type=jnp.float32) + b_ref[...]


def _combine_kernel(p_ref, r_ref, o_ref):
    o_ref[...] = jax.nn.relu(p_ref[0] + p_ref[1] + r_ref[...])


_ROWB = 2000


def _mm(x, Wcat, bcat):
    f = x.shape[1]
    c = Wcat.shape[1]
    return pl.pallas_call(
        _mm_kernel,
        grid=(N // _ROWB,),
        in_specs=[pl.BlockSpec((_ROWB, f), lambda i: (i, 0)),
                  pl.BlockSpec((f, c), lambda i: (0, 0)),
                  pl.BlockSpec((1, c), lambda i: (0, 0))],
        out_specs=pl.BlockSpec((_ROWB, c), lambda i: (i, 0)),
        out_shape=jax.ShapeDtypeStruct((N, c), jnp.float32),
    )(x, Wcat, bcat)


def _combine_mm(parts, R, skip, Wcat, bcat):
    f = Wcat.shape[0]
    c = Wcat.shape[1]
    specs = [pl.BlockSpec((2, _ROWB, OUT), lambda i: (0, i, 0)),
             pl.BlockSpec((_ROWB, OUT), lambda i: (i, 0))]
    args = [parts, R]
    if skip is not None:
        specs.append(pl.BlockSpec((_ROWB, OUT), lambda i: (i, 0)))
        args.append(skip)
    specs += [pl.BlockSpec((f, c), lambda i: (0, 0)),
              pl.BlockSpec((1, c), lambda i: (0, 0))]
    args += [Wcat, bcat]

    def body(*refs):
        if skip is not None:
            p_ref, r_ref, s_ref, w_ref, b_ref, o_ref = refs
            h = jax.nn.relu(p_ref[0] + p_ref[1] + r_ref[...])
            h = jnp.concatenate([h, s_ref[...]], axis=1)
        else:
            p_ref, r_ref, w_ref, b_ref, o_ref = refs
            h = jax.nn.relu(p_ref[0] + p_ref[1] + r_ref[...])
        o_ref[...] = jax.lax.dot_general(
            h, w_ref[...], (((1,), (0,)), ((), ())),
            preferred_element_type=jnp.float32) + b_ref[...]

    return pl.pallas_call(
        body,
        grid=(N // _ROWB,),
        in_specs=specs,
        out_specs=pl.BlockSpec((_ROWB, c), lambda i: (i, 0)),
        out_shape=jax.ShapeDtypeStruct((N, c), jnp.float32),
    )(*args)


def _combine(parts, R):
    return pl.pallas_call(
        _combine_kernel,
        grid=(N // _ROWB,),
        in_specs=[pl.BlockSpec((2, _ROWB, OUT), lambda i: (0, i, 0)),
                  pl.BlockSpec((_ROWB, OUT), lambda i: (i, 0))],
        out_specs=pl.BlockSpec((_ROWB, OUT), lambda i: (i, 0)),
        out_shape=jax.ShapeDtypeStruct((N, OUT), jnp.float32),
    )(parts, R)


_B = 80
_NTILE = 16
_NPAD = 10240
_ROWS_PER_TILE = _NPAD // _NTILE


def _sc_conv(H, basis, src, dst, zeros):
    kc = H.shape[1]
    e_per_w = E // 32
    n_chunks = e_per_w // _B
    mesh = plsc.VectorSubcoreMesh(core_axis_name="c", subcore_axis_name="s",
                                  num_cores=2, num_subcores=_NTILE)

    @functools.partial(
        pl.kernel,
        out_type=jax.ShapeDtypeStruct((2, _NPAD, OUT), jnp.float32),
        mesh=mesh,
        compiler_params=pltpu.CompilerParams(use_tc_tiling_on_sc=False),
        scratch_types=[
            pltpu.VMEM((_B,), jnp.int32),
            pltpu.VMEM((_B,), jnp.int32),
            pltpu.VMEM((_B, 16), jnp.float32),
            pltpu.VMEM((_B, kc), jnp.float32),
            pltpu.VMEM((_B, OUT), jnp.float32),
            pltpu.VMEM_SHARED((_NPAD, OUT), jnp.float32),
            pltpu.SemaphoreType.DMA,
        ],
    )
    def k(h_hbm, basis_hbm, src_hbm, dst_hbm, z_hbm, out_hbm,
          sidx, didx, bas, rows, msg, acc, gsem):
        c = lax.axis_index("c")
        s = lax.axis_index("s")
        wid = c * _NTILE + s

        r0 = s * _ROWS_PER_TILE
        pltpu.sync_copy(z_hbm.at[pl.ds(r0, _ROWS_PER_TILE)],
                        acc.at[pl.ds(r0, _ROWS_PER_TILE)])
        plsc.subcore_barrier()

        ebase = wid * e_per_w

        @pl.loop(0, n_chunks)
        def chunk(ci):
            base = ebase + ci * _B
            pltpu.sync_copy(src_hbm.at[pl.ds(base, _B)], sidx)
            pltpu.sync_copy(dst_hbm.at[pl.ds(base, _B)], didx)
            pltpu.sync_copy(basis_hbm.at[pl.ds(base, _B)], bas)
            pltpu.async_copy(h_hbm.at[sidx], rows, gsem).wait()

            @pl.loop(0, _B)
            def per_edge(b):
                wv = bas[b]
                w = [wv[kk] for kk in range(K)]
                for oc in range(OUT // 16):
                    acc16 = w[0] * rows[b, pl.ds(oc * 16, 16)]
                    for kk in range(1, K):
                        acc16 = acc16 + w[kk] * rows[b, pl.ds(kk * OUT + oc * 16, 16)]
                    msg[b, pl.ds(oc * 16, 16)] = acc16

            pltpu.sync_copy(msg, acc.at[didx], add=True)

        plsc.subcore_barrier()
        pltpu.sync_copy(acc.at[pl.ds(r0, _ROWS_PER_TILE)],
                        out_hbm.at[c, pl.ds(r0, _ROWS_PER_TILE)])

    return k(H, basis, src, dst, zeros)[:, :N, :]


def kernel(x, edge_index, edge_attr, skip, W1, root1, b1, W2, root2, b2):
    src = edge_index[0]
    dst = edge_index[1]
    basis = _compute_basis(edge_attr)
    zeros = jnp.zeros((_NPAD, OUT), jnp.float32)

    W1cat = jnp.concatenate([W1.transpose(1, 0, 2).reshape(IN, K * OUT),
                             root1], axis=1)
    b1cat = jnp.concatenate([jnp.zeros((K * OUT,), jnp.float32), b1])[None, :]
    W2cat = jnp.concatenate([W2.transpose(1, 0, 2).reshape(OUT, K * OUT),
                             root2], axis=1)
    b2cat = jnp.concatenate([jnp.zeros((K * OUT,), jnp.float32), b2])[None, :]

    HR1 = _mm(x, W1cat, b1cat)
    parts1 = _sc_conv(HR1, basis, src, dst, zeros)
    HR2 = _combine_mm(parts1, HR1[:, K * OUT:], skip, W1cat, b1cat)
    parts2 = _sc_conv(HR2, basis, src, dst, zeros)
    HR3 = _combine_mm(parts2, HR2[:, K * OUT:], None, W2cat, b2cat)
    parts3 = _sc_conv(HR3, basis, src, dst, zeros)
    return _combine(parts3, HR3[:, K * OUT:])


# speedup vs baseline: 2.9796x; 1.1333x over previous
import functools

import jax
import jax.numpy as jnp
from jax import lax
from jax.experimental import pallas as pl
from jax.experimental.pallas import tpu as pltpu
from jax.experimental.pallas import tpu_sc as plsc

N = 10000
E = 320000
IN = 128
OUT = 64
KS = 3
K = KS * KS


def _basis_planes(a0, a1):
    def comp(u):
        planes = []
        for j in range(KS):
            t = u - jnp.float32(j)
            at = jnp.abs(t)
            b = jnp.where(at <= 0.5, 0.75 - t * t,
                          jnp.where(at <= 1.5, 0.5 * (1.5 - at) ** 2,
                                    jnp.float32(0.0)))
            planes.append(b)
        s = planes[0] + planes[1] + planes[2] + jnp.float32(1e-12)
        return [p / s for p in planes]

    b0 = comp(a0 * jnp.float32(KS - 1))
    b1 = comp(a1 * jnp.float32(KS - 1))
    return [b0[i] * b1[j] for i in range(KS) for j in range(KS)]


def _basis_kernel(a0_ref, a1_ref, *out_refs):
    planes = _basis_planes(a0_ref[...], a1_ref[...])
    for r, p in zip(out_refs, planes):
        r[...] = p


def _compute_basis(edge_attr):
    rows = E // 128
    a0 = edge_attr[:, 0].reshape(rows, 128)
    a1 = edge_attr[:, 1].reshape(rows, 128)
    outs = pl.pallas_call(
        _basis_kernel,
        out_shape=[jax.ShapeDtypeStruct((rows, 128), jnp.float32)] * K,
    )(a0, a1)
    z = jnp.zeros_like(outs[0])
    return jnp.stack(list(outs) + [z] * (16 - K), axis=-1).reshape(E, 16)


def _mm_kernel(x_ref, w_ref, b_ref, o_ref):
    o_ref[...] = jax.lax.dot_general(
        x_ref[...], w_ref[...], (((1,), (0,)), ((), ())),
        preferred_element_type=jnp.float32) + b_ref[...]


def _combine_kernel(p_ref, r_ref, o_ref):
    o_ref[...] = jax.nn.relu(p_ref[0] + p_ref[1] + r_ref[...])


_ROWB = 2000


def _mm(x, Wcat, bcat):
    f = x.shape[1]
    c = Wcat.shape[1]
    return pl.pallas_call(
        _mm_kernel,
        grid=(N // _ROWB,),
        in_specs=[pl.BlockSpec((_ROWB, f), lambda i: (i, 0)),
                  pl.BlockSpec((f, c), lambda i: (0, 0)),
                  pl.BlockSpec((1, c), lambda i: (0, 0))],
        out_specs=pl.BlockSpec((_ROWB, c), lambda i: (i, 0)),
        out_shape=jax.ShapeDtypeStruct((N, c), jnp.float32),
    )(x, Wcat, bcat)


def _combine_mm(parts, R, skip, Wcat, bcat):
    f = Wcat.shape[0]
    c = Wcat.shape[1]
    specs = [pl.BlockSpec((2, _ROWB, OUT), lambda i: (0, i, 0)),
             pl.BlockSpec((_ROWB, OUT), lambda i: (i, 0))]
    args = [parts, R]
    if skip is not None:
        specs.append(pl.BlockSpec((_ROWB, OUT), lambda i: (i, 0)))
        args.append(skip)
    specs += [pl.BlockSpec((f, c), lambda i: (0, 0)),
              pl.BlockSpec((1, c), lambda i: (0, 0))]
    args += [Wcat, bcat]

    def body(*refs):
        if skip is not None:
            p_ref, r_ref, s_ref, w_ref, b_ref, o_ref = refs
            h = jax.nn.relu(p_ref[0] + p_ref[1] + r_ref[...])
            h = jnp.concatenate([h, s_ref[...]], axis=1)
        else:
            p_ref, r_ref, w_ref, b_ref, o_ref = refs
            h = jax.nn.relu(p_ref[0] + p_ref[1] + r_ref[...])
        o_ref[...] = jax.lax.dot_general(
            h, w_ref[...], (((1,), (0,)), ((), ())),
            preferred_element_type=jnp.float32) + b_ref[...]

    return pl.pallas_call(
        body,
        grid=(N // _ROWB,),
        in_specs=specs,
        out_specs=pl.BlockSpec((_ROWB, c), lambda i: (i, 0)),
        out_shape=jax.ShapeDtypeStruct((N, c), jnp.float32),
    )(*args)


def _combine(parts, R):
    return pl.pallas_call(
        _combine_kernel,
        grid=(N // _ROWB,),
        in_specs=[pl.BlockSpec((2, _ROWB, OUT), lambda i: (0, i, 0)),
                  pl.BlockSpec((_ROWB, OUT), lambda i: (i, 0))],
        out_specs=pl.BlockSpec((_ROWB, OUT), lambda i: (i, 0)),
        out_shape=jax.ShapeDtypeStruct((N, OUT), jnp.float32),
    )(parts, R)


_B = 40
_NTILE = 16
_NPAD = 10240
_ROWS_PER_TILE = _NPAD // _NTILE


def _sc_conv(H, basis, src, dst, zeros):
    kc = H.shape[1]
    e_per_w = E // 32
    n_chunks = e_per_w // _B
    mesh = plsc.VectorSubcoreMesh(core_axis_name="c", subcore_axis_name="s",
                                  num_cores=2, num_subcores=_NTILE)

    @functools.partial(
        pl.kernel,
        out_type=jax.ShapeDtypeStruct((2, _NPAD, OUT), jnp.float32),
        mesh=mesh,
        compiler_params=pltpu.CompilerParams(use_tc_tiling_on_sc=False),
        scratch_types=[
            [pltpu.VMEM((_B,), jnp.int32)] * 2,
            [pltpu.VMEM((_B,), jnp.int32)] * 2,
            [pltpu.VMEM((_B, 16), jnp.float32)] * 2,
            [pltpu.VMEM((_B, kc), jnp.float32)] * 2,
            pltpu.VMEM((_B, OUT), jnp.float32),
            pltpu.VMEM_SHARED((_NPAD, OUT), jnp.float32),
            [pltpu.SemaphoreType.DMA] * 2,
        ],
    )
    def k(h_hbm, basis_hbm, src_hbm, dst_hbm, z_hbm, out_hbm,
          sidx, didx, bas, rows, msg, acc, gsem):
        c = lax.axis_index("c")
        s = lax.axis_index("s")
        wid = c * _NTILE + s

        r0 = s * _ROWS_PER_TILE
        pltpu.sync_copy(z_hbm.at[pl.ds(r0, _ROWS_PER_TILE)],
                        acc.at[pl.ds(r0, _ROWS_PER_TILE)])
        plsc.subcore_barrier()

        ebase = wid * e_per_w

        def prefetch(ci, j):
            base = ebase + ci * _B
            pltpu.sync_copy(src_hbm.at[pl.ds(base, _B)], sidx[j])
            pltpu.sync_copy(dst_hbm.at[pl.ds(base, _B)], didx[j])
            pltpu.sync_copy(basis_hbm.at[pl.ds(base, _B)], bas[j])
            pltpu.async_copy(h_hbm.at[sidx[j]], rows[j], gsem[j])

        def consume(j):
            pltpu.make_async_copy(h_hbm.at[sidx[j]], rows[j], gsem[j]).wait()

            @pl.loop(0, _B)
            def per_edge(b):
                wv = bas[j][b]
                w = [wv[kk] for kk in range(K)]
                for oc in range(OUT // 16):
                    acc16 = w[0] * rows[j][b, pl.ds(oc * 16, 16)]
                    for kk in range(1, K):
                        acc16 = acc16 + w[kk] * rows[j][b, pl.ds(kk * OUT + oc * 16, 16)]
                    msg[b, pl.ds(oc * 16, 16)] = acc16

            pltpu.sync_copy(msg, acc.at[didx[j]], add=True)

        prefetch(0, 0)

        @pl.loop(0, (n_chunks - 2) // 2)
        def pair(pi):
            prefetch(2 * pi + 1, 1)
            consume(0)
            prefetch(2 * pi + 2, 0)
            consume(1)

        prefetch(n_chunks - 1, 1)
        consume(0)
        consume(1)

        plsc.subcore_barrier()
        pltpu.sync_copy(acc.at[pl.ds(r0, _ROWS_PER_TILE)],
                        out_hbm.at[c, pl.ds(r0, _ROWS_PER_TILE)])

    return k(H, basis, src, dst, zeros)[:, :N, :]


def kernel(x, edge_index, edge_attr, skip, W1, root1, b1, W2, root2, b2):
    src = edge_index[0]
    dst = edge_index[1]
    basis = _compute_basis(edge_attr)
    zeros = jnp.zeros((_NPAD, OUT), jnp.float32)

    W1cat = jnp.concatenate([W1.transpose(1, 0, 2).reshape(IN, K * OUT),
                             root1], axis=1)
    b1cat = jnp.concatenate([jnp.zeros((K * OUT,), jnp.float32), b1])[None, :]
    W2cat = jnp.concatenate([W2.transpose(1, 0, 2).reshape(OUT, K * OUT),
                             root2], axis=1)
    b2cat = jnp.concatenate([jnp.zeros((K * OUT,), jnp.float32), b2])[None, :]

    HR1 = _mm(x, W1cat, b1cat)
    parts1 = _sc_conv(HR1[:, :K * OUT], basis, src, dst, zeros)
    HR2 = _combine_mm(parts1, HR1[:, K * OUT:], skip, W1cat, b1cat)
    parts2 = _sc_conv(HR2[:, :K * OUT], basis, src, dst, zeros)
    HR3 = _combine_mm(parts2, HR2[:, K * OUT:], None, W2cat, b2cat)
    parts3 = _sc_conv(HR3[:, :K * OUT], basis, src, dst, zeros)
    return _combine(parts3, HR3[:, K * OUT:])
